# baseline (speedup 1.0000x reference)
import functools

import jax
import jax.numpy as jnp
from jax import lax
from jax.experimental import pallas as pl
from jax.experimental.pallas import tpu as pltpu
from jax.experimental.pallas import tpu_sc as plsc

_VOCAB = 100000
_HID = 768
_MAXS = 2048
_B = 4
_EPS = 1e-12

_L = 16
_HL = _HID // _L
_NW = 32
_N = _B * _MAXS
_TPW = _N // _NW
_C = 64
_NCH = _TPW // _C


def _rsqrt16(x):
    i = plsc.bitcast(x, jnp.int32)
    i = jnp.int32(0x5F3759DF) - lax.shift_right_arithmetic(i, jnp.int32(1))
    y = plsc.bitcast(i, jnp.float32)
    half = x * 0.5
    for _ in range(3):
        y = y * (1.5 - half * y * y)
    return y


def _sc_body(ids_hbm, sid_hbm, tok_hbm, pos_hbm, seg_hbm, gam_hbm, bet_hbm,
             out_hbm,
             ids_v, sid_v, tokbuf, posbuf, segbuf, gam_v, bet_v, sem):
    wid = lax.axis_index("s") * 2 + lax.axis_index("c")
    base = wid * _TPW
    s0 = lax.rem(wid, 8) * _TPW

    pltpu.sync_copy(seg_hbm, segbuf)
    pltpu.sync_copy(gam_hbm, gam_v)
    pltpu.sync_copy(bet_hbm, bet_v)

    inv_h = jnp.float32(1.0 / _HID)

    for g in range(_NCH):
        cbase = base + g * _C
        pltpu.sync_copy(ids_hbm.at[wid, g], ids_v)
        gather = pltpu.async_copy(tok_hbm.at[ids_v], tokbuf, sem)
        pltpu.sync_copy(sid_hbm.at[wid, g], sid_v)
        pltpu.sync_copy(pos_hbm.at[pl.ds(s0 + g * _C, _C)], posbuf)
        gather.wait()

        def token_body(t, _):
            sidf = sid_v[pl.ds(t, _L)][0].astype(jnp.float32)

            def pass1(h, carry):
                s1, s2 = carry
                ds = pl.ds(h * _L, _L)
                e = (tokbuf[t, ds] + posbuf[t, ds]
                     + segbuf[0, ds] * (1.0 - sidf) + segbuf[1, ds] * sidf)
                tokbuf[t, ds] = e
                return s1 + e, s2 + e * e

            z = jnp.zeros((_L,), jnp.float32)
            s1, s2 = lax.fori_loop(0, _HL, pass1, (z, z))
            mean = jnp.sum(s1) * inv_h
            var = jnp.sum(s2) * inv_h - mean * mean
            rstd = _rsqrt16(jnp.full((_L,), var + _EPS, jnp.float32))
            mvec = jnp.full((_L,), mean, jnp.float32)

            def pass2(h, _):
                ds = pl.ds(h * _L, _L)
                e = tokbuf[t, ds]
                tokbuf[t, ds] = ((e - mvec) * rstd) * gam_v[ds] + bet_v[ds]
                return 0

            lax.fori_loop(0, _HL, pass2, 0)
            return 0

        lax.fori_loop(0, _C, token_body, 0)
        pltpu.sync_copy(tokbuf, out_hbm.at[pl.ds(cbase, _C)])


@jax.jit
def _run(ids_w, sid_w, token_table, pos_table, seg_table, gamma, beta):
    mesh = plsc.VectorSubcoreMesh(core_axis_name="c", subcore_axis_name="s")
    k = pl.kernel(
        _sc_body,
        out_type=jax.ShapeDtypeStruct((_N, _HID), jnp.float32),
        mesh=mesh,
        compiler_params=pltpu.CompilerParams(needs_layout_passes=False),
        scratch_types=[
            pltpu.VMEM((_C,), jnp.int32),
            pltpu.VMEM((_C + _L,), jnp.int32),
            pltpu.VMEM((_C, _HID), jnp.float32),
            pltpu.VMEM((_C, _HID), jnp.float32),
            pltpu.VMEM((2, _HID), jnp.float32),
            pltpu.VMEM((_HID,), jnp.float32),
            pltpu.VMEM((_HID,), jnp.float32),
            pltpu.SemaphoreType.DMA,
        ],
    )
    return k(ids_w, sid_w, token_table, pos_table, seg_table, gamma, beta)


def kernel(input_ids, segment_ids, token_table, pos_table, seg_table, gamma, beta):
    ids_w = input_ids.reshape(_NW, _NCH, _C).astype(jnp.int32)
    sid_w = segment_ids.reshape(_NW, _NCH, _C).astype(jnp.int32)
    sid_w = jnp.pad(sid_w, ((0, 0), (0, 0), (0, _L)))
    out = _run(ids_w, sid_w, token_table, pos_table, seg_table, gamma, beta)
    return out.reshape(_B, _MAXS, _HID)


# speedup vs baseline: 1.5294x; 1.5294x over previous
import jax
import jax.numpy as jnp
from jax import lax
from jax.experimental import pallas as pl
from jax.experimental.pallas import tpu as pltpu
from jax.experimental.pallas import tpu_sc as plsc

_VOCAB = 100000
_HID = 768
_MAXS = 2048
_B = 4
_EPS = 1e-12

_L = 16
_HL = _HID // _L
_NW = 32
_N = _B * _MAXS
_TPW = _N // _NW
_C = 64
_NCH = _TPW // _C


def _rsqrt16(x):
    i = plsc.bitcast(x, jnp.int32)
    i = jnp.int32(0x5F3759DF) - lax.shift_right_arithmetic(i, jnp.int32(1))
    y = plsc.bitcast(i, jnp.float32)
    half = x * 0.5
    for _ in range(3):
        y = y * (1.5 - half * y * y)
    return y


def _sc_body(ids_hbm, sid_hbm, tok_hbm, pos_hbm, seg_hbm, gam_hbm, bet_hbm,
             out_hbm,
             ids_v, sid_v, tokbuf, posbuf, segbuf, dseg_v, gam_v, bet_v, sem):
    wid = lax.axis_index("s") * 2 + lax.axis_index("c")
    base = wid * _TPW
    s0 = lax.rem(wid, 8) * _TPW

    pltpu.sync_copy(seg_hbm, segbuf)
    pltpu.sync_copy(gam_hbm, gam_v)
    pltpu.sync_copy(bet_hbm, bet_v)
    for h in range(_HL):
        ds = pl.ds(h * _L, _L)
        dseg_v[ds] = segbuf[1, ds] - segbuf[0, ds]

    inv_h = jnp.float32(1.0 / _HID)

    def chunk_body(g, _):
        cbase = base + g * _C
        pltpu.sync_copy(ids_hbm.at[wid, g], ids_v)
        gather = pltpu.async_copy(tok_hbm.at[ids_v], tokbuf, sem)
        pltpu.sync_copy(sid_hbm.at[wid, g], sid_v)
        pltpu.sync_copy(pos_hbm.at[pl.ds(s0 + g * _C, _C)], posbuf)
        gather.wait()

        def pair_body(p, _):
            t0 = 2 * p
            t1 = t0 + 1
            sid2 = sid_v[pl.ds(t0, _L)]
            sidf0 = sid2[0].astype(jnp.float32)
            sidf1 = sid2[1].astype(jnp.float32)

            z = jnp.zeros((_L,), jnp.float32)
            sa1 = sa2 = sb1 = sb2 = z
            for h in range(_HL):
                ds = pl.ds(h * _L, _L)
                s0v = segbuf[0, ds]
                dsv = dseg_v[ds]
                ea = tokbuf[t0, ds] + posbuf[t0, ds] + (s0v + sidf0 * dsv)
                eb = tokbuf[t1, ds] + posbuf[t1, ds] + (s0v + sidf1 * dsv)
                tokbuf[t0, ds] = ea
                tokbuf[t1, ds] = eb
                sa1 = sa1 + ea
                sa2 = sa2 + ea * ea
                sb1 = sb1 + eb
                sb2 = sb2 + eb * eb

            ma = jnp.sum(sa1) * inv_h
            mb = jnp.sum(sb1) * inv_h
            va = jnp.sum(sa2) * inv_h - ma * ma
            vb = jnp.sum(sb2) * inv_h - mb * mb
            ra = _rsqrt16(jnp.full((_L,), va + _EPS, jnp.float32))
            rb = _rsqrt16(jnp.full((_L,), vb + _EPS, jnp.float32))
            mav = jnp.full((_L,), ma, jnp.float32)
            mbv = jnp.full((_L,), mb, jnp.float32)

            for h in range(_HL):
                ds = pl.ds(h * _L, _L)
                gv = gam_v[ds]
                bv = bet_v[ds]
                ea = tokbuf[t0, ds]
                eb = tokbuf[t1, ds]
                tokbuf[t0, ds] = ((ea - mav) * ra) * gv + bv
                tokbuf[t1, ds] = ((eb - mbv) * rb) * gv + bv
            return 0

        lax.fori_loop(0, _C // 2, pair_body, 0)
        pltpu.sync_copy(tokbuf, out_hbm.at[pl.ds(cbase, _C)])
        return 0

    lax.fori_loop(0, _NCH, chunk_body, 0)


@jax.jit
def _run(ids_w, sid_w, token_table, pos_table, seg_table, gamma, beta):
    mesh = plsc.VectorSubcoreMesh(core_axis_name="c", subcore_axis_name="s")
    k = pl.kernel(
        _sc_body,
        out_type=jax.ShapeDtypeStruct((_N, _HID), jnp.float32),
        mesh=mesh,
        compiler_params=pltpu.CompilerParams(needs_layout_passes=False),
        scratch_types=[
            pltpu.VMEM((_C,), jnp.int32),
            pltpu.VMEM((_C + _L,), jnp.int32),
            pltpu.VMEM((_C, _HID), jnp.float32),
            pltpu.VMEM((_C, _HID), jnp.float32),
            pltpu.VMEM((2, _HID), jnp.float32),
            pltpu.VMEM((_HID,), jnp.float32),
            pltpu.VMEM((_HID,), jnp.float32),
            pltpu.VMEM((_HID,), jnp.float32),
            pltpu.SemaphoreType.DMA,
        ],
    )
    return k(ids_w, sid_w, token_table, pos_table, seg_table, gamma, beta)


def kernel(input_ids, segment_ids, token_table, pos_table, seg_table, gamma, beta):
    ids_w = input_ids.reshape(_NW, _NCH, _C).astype(jnp.int32)
    sid_w = segment_ids.reshape(_NW, _NCH, _C).astype(jnp.int32)
    sid_w = jnp.pad(sid_w, ((0, 0), (0, 0), (0, _L)))
    out = _run(ids_w, sid_w, token_table, pos_table, seg_table, gamma, beta)
    return out.reshape(_B, _MAXS, _HID)


# speedup vs baseline: 3.9536x; 2.5850x over previous
import jax
import jax.numpy as jnp
from jax import lax
from jax.experimental import pallas as pl
from jax.experimental.pallas import tpu as pltpu
from jax.experimental.pallas import tpu_sc as plsc

_VOCAB = 100000
_HID = 768
_MAXS = 2048
_B = 4
_EPS = 1e-12

_NW = 32
_N = _B * _MAXS
_TPW = _N // _NW
_C = 64
_NCH = _TPW // _C

_BN = 512
_NBLK = _N // _BN
_SPB = _MAXS // _BN


def _sc_gather_body(ids_hbm, tok_hbm, out_hbm,
                    ids0, ids1, buf0, buf1, gsem0, gsem1, wsem0, wsem1):
    wid = lax.axis_index("s") * 2 + lax.axis_index("c")
    base = wid * _TPW
    idbufs = (ids0, ids1)
    bufs = (buf0, buf1)
    gsems = (gsem0, gsem1)
    wsems = (wsem0, wsem1)

    pltpu.sync_copy(ids_hbm.at[wid, 0], ids0)
    gathers = [pltpu.async_copy(tok_hbm.at[ids0], buf0, gsem0), None]
    writes = [None, None]
    for g in range(_NCH):
        p = g % 2
        np_ = (g + 1) % 2
        if g + 1 < _NCH:
            if writes[np_] is not None:
                writes[np_].wait()
                writes[np_] = None
            pltpu.sync_copy(ids_hbm.at[wid, g + 1], idbufs[np_])
            gathers[np_] = pltpu.async_copy(
                tok_hbm.at[idbufs[np_]], bufs[np_], gsems[np_])
        gathers[p].wait()
        writes[p] = pltpu.async_copy(
            bufs[p], out_hbm.at[pl.ds(base + g * _C, _C)], wsems[p])
    for p in range(2):
        if writes[p] is not None:
            writes[p].wait()


def _tc_ln_body(emb_ref, pos_ref, sid_ref, seg_ref, gam_ref, bet_ref, out_ref):
    e = emb_ref[...] + pos_ref[...]
    sidf = sid_ref[0]
    s0 = seg_ref[0:1, :]
    s1 = seg_ref[1:2, :]
    e = e + s0 + sidf.reshape(_BN, 1) * (s1 - s0)
    mean = jnp.mean(e, axis=-1, keepdims=True)
    var = jnp.mean((e - mean) ** 2, axis=-1, keepdims=True)
    normed = (e - mean) * lax.rsqrt(var + _EPS)
    out_ref[...] = normed * gam_ref[...].reshape(1, _HID) + bet_ref[...].reshape(1, _HID)


@jax.jit
def _run(ids_w, sidf3, token_table, pos_table, seg_table, gamma, beta):
    mesh = plsc.VectorSubcoreMesh(core_axis_name="c", subcore_axis_name="s")
    gath = pl.kernel(
        _sc_gather_body,
        out_type=jax.ShapeDtypeStruct((_N, _HID), jnp.float32),
        mesh=mesh,
        compiler_params=pltpu.CompilerParams(needs_layout_passes=False),
        scratch_types=[
            pltpu.VMEM((_C,), jnp.int32),
            pltpu.VMEM((_C,), jnp.int32),
            pltpu.VMEM((_C, _HID), jnp.float32),
            pltpu.VMEM((_C, _HID), jnp.float32),
            pltpu.SemaphoreType.DMA,
            pltpu.SemaphoreType.DMA,
            pltpu.SemaphoreType.DMA,
            pltpu.SemaphoreType.DMA,
        ],
    )
    rows = gath(ids_w, token_table)

    out = pl.pallas_call(
        _tc_ln_body,
        out_shape=jax.ShapeDtypeStruct((_N, _HID), jnp.float32),
        grid=(_NBLK,),
        in_specs=[
            pl.BlockSpec((_BN, _HID), lambda g: (g, 0)),
            pl.BlockSpec((_BN, _HID), lambda g: (g % _SPB, 0)),
            pl.BlockSpec((1, 1, _BN), lambda g: (g, 0, 0)),
            pl.BlockSpec((2, _HID), lambda g: (0, 0)),
            pl.BlockSpec((_HID,), lambda g: (0,)),
            pl.BlockSpec((_HID,), lambda g: (0,)),
        ],
        out_specs=pl.BlockSpec((_BN, _HID), lambda g: (g, 0)),
    )(rows, pos_table, sidf3, seg_table, gamma, beta)
    return out


def kernel(input_ids, segment_ids, token_table, pos_table, seg_table, gamma, beta):
    ids_w = input_ids.reshape(_NW, _NCH, _C).astype(jnp.int32)
    sidf3 = segment_ids.reshape(_NBLK, 1, _BN).astype(jnp.float32)
    out = _run(ids_w, sidf3, token_table, pos_table, seg_table, gamma, beta)
    return out.reshape(_B, _MAXS, _HID)


# speedup vs baseline: 4.1249x; 1.0433x over previous
import jax
import jax.numpy as jnp
from jax import lax
from jax.experimental import pallas as pl
from jax.experimental.pallas import tpu as pltpu
from jax.experimental.pallas import tpu_sc as plsc

_VOCAB = 100000
_HID = 768
_MAXS = 2048
_B = 4
_EPS = 1e-12

_NW = 32
_N = _B * _MAXS
_TPW = _N // _NW
_C = 64
_NCH = _TPW // _C

_BN = 512
_NBLK = _N // _BN
_SPB = _MAXS // _BN


def _sc_gather_body(ids_hbm, tok_hbm, out_hbm,
                    ids0, ids1, buf0, buf1, gsem0, gsem1, wsem0, wsem1):
    wid = lax.axis_index("s") * 2 + lax.axis_index("c")
    base = wid * _TPW
    idbufs = (ids0, ids1)
    bufs = (buf0, buf1)
    gsems = (gsem0, gsem1)
    wsems = (wsem0, wsem1)

    pltpu.sync_copy(ids_hbm.at[wid, 0], ids0)
    gathers = [pltpu.async_copy(tok_hbm.at[ids0], buf0, gsem0), None]
    writes = [None, None]
    for g in range(_NCH):
        p = g % 2
        np_ = (g + 1) % 2
        if g + 1 < _NCH:
            if writes[np_] is not None:
                writes[np_].wait()
                writes[np_] = None
            pltpu.sync_copy(ids_hbm.at[wid, g + 1], idbufs[np_])
            gathers[np_] = pltpu.async_copy(
                tok_hbm.at[idbufs[np_]], bufs[np_], gsems[np_])
        gathers[p].wait()
        writes[p] = pltpu.async_copy(
            bufs[p], out_hbm.at[pl.ds(base + g * _C, _C)], wsems[p])
    for p in range(2):
        if writes[p] is not None:
            writes[p].wait()


def _tc_ln_body(emb_ref, pos_ref, sid_ref, seg_ref, gam_ref, bet_ref, out_ref):
    g = pl.program_id(0)
    prow = lax.rem(g, _SPB) * _BN
    e = emb_ref[...] + pos_ref[pl.ds(prow, _BN), :]
    sidf = sid_ref[0]
    s0 = seg_ref[0:1, :]
    s1 = seg_ref[1:2, :]
    e = e + s0 + sidf.reshape(_BN, 1) * (s1 - s0)
    mean = jnp.mean(e, axis=-1, keepdims=True)
    var = jnp.mean((e - mean) ** 2, axis=-1, keepdims=True)
    normed = (e - mean) * lax.rsqrt(var + _EPS)
    out_ref[...] = normed * gam_ref[...].reshape(1, _HID) + bet_ref[...].reshape(1, _HID)


@jax.jit
def _run(ids_w, sidf3, token_table, pos_table, seg_table, gamma, beta):
    mesh = plsc.VectorSubcoreMesh(core_axis_name="c", subcore_axis_name="s")
    gath = pl.kernel(
        _sc_gather_body,
        out_type=jax.ShapeDtypeStruct((_N, _HID), jnp.float32),
        mesh=mesh,
        compiler_params=pltpu.CompilerParams(needs_layout_passes=False),
        scratch_types=[
            pltpu.VMEM((_C,), jnp.int32),
            pltpu.VMEM((_C,), jnp.int32),
            pltpu.VMEM((_C, _HID), jnp.float32),
            pltpu.VMEM((_C, _HID), jnp.float32),
            pltpu.SemaphoreType.DMA,
            pltpu.SemaphoreType.DMA,
            pltpu.SemaphoreType.DMA,
            pltpu.SemaphoreType.DMA,
        ],
    )
    rows = gath(ids_w, token_table)

    out = pl.pallas_call(
        _tc_ln_body,
        out_shape=jax.ShapeDtypeStruct((_N, _HID), jnp.float32),
        grid=(_NBLK,),
        in_specs=[
            pl.BlockSpec((_BN, _HID), lambda g: (g, 0)),
            pl.BlockSpec((_MAXS, _HID), lambda g: (0, 0)),
            pl.BlockSpec((1, 1, _BN), lambda g: (g, 0, 0)),
            pl.BlockSpec((2, _HID), lambda g: (0, 0)),
            pl.BlockSpec((_HID,), lambda g: (0,)),
            pl.BlockSpec((_HID,), lambda g: (0,)),
        ],
        out_specs=pl.BlockSpec((_BN, _HID), lambda g: (g, 0)),
    )(rows, pos_table, sidf3, seg_table, gamma, beta)
    return out


def kernel(input_ids, segment_ids, token_table, pos_table, seg_table, gamma, beta):
    ids_w = input_ids.reshape(_NW, _NCH, _C).astype(jnp.int32)
    sidf3 = segment_ids.reshape(_NBLK, 1, _BN).astype(jnp.float32)
    out = _run(ids_w, sidf3, token_table, pos_table, seg_table, gamma, beta)
    return out.reshape(_B, _MAXS, _HID)
